# Initial kernel scaffold; baseline (speedup 1.0000x reference)
import jax
import jax.numpy as jnp
from jax.experimental import pallas as pl


def kernel(epoch, inputs, ema_inputs, part_out, score, targets, features, part_features):
    raise NotImplementedError("write your pallas kernel here")


# speedup vs baseline: 2.2394x; 2.2394x over previous
import functools

import jax
import jax.numpy as jnp
from jax.experimental import pallas as pl
from jax.experimental.pallas import tpu as pltpu

_TEMP = 0.05
_BN = 512


def _ce_kernel(x_ref, f_ref, t_ref, out_ref, xb_ref, s_ref, tacc_ref):
    j = pl.program_id(0)
    nj = pl.num_programs(0)
    bn = f_ref.shape[0]

    @pl.when(j == 0)
    def _init():
        x = x_ref[...]
        norm = jnp.sqrt(jnp.sum(x * x, axis=1, keepdims=True))
        xn = x / jnp.maximum(norm, 1e-12)
        xb_ref[...] = xn.astype(jnp.bfloat16)
        s_ref[...] = jnp.zeros_like(s_ref)
        tacc_ref[...] = jnp.zeros_like(tacc_ref)

    fb = f_ref[...].astype(jnp.bfloat16)
    logits = jax.lax.dot_general(
        xb_ref[...], fb, (((1,), (1,)), ((), ())),
        preferred_element_type=jnp.float32,
    ) * (1.0 / _TEMP)
    s_ref[...] += jnp.sum(jnp.exp(logits), axis=1, keepdims=True)
    cols = j * bn + jax.lax.broadcasted_iota(jnp.int32, logits.shape, 1)
    hit = cols == t_ref[...]
    tacc_ref[...] += jnp.sum(jnp.where(hit, logits, 0.0), axis=1, keepdims=True)

    @pl.when(j == nj - 1)
    def _fin():
        per_row = jnp.log(s_ref[...]) - tacc_ref[...]
        out_ref[...] = jnp.sum(per_row, keepdims=True) * (1.0 / per_row.shape[0])


def kernel(epoch, inputs, ema_inputs, part_out, score, targets, features,
           part_features):
    m, k = inputs.shape
    n = features.shape[0]
    out = pl.pallas_call(
        _ce_kernel,
        grid=(n // _BN,),
        in_specs=[
            pl.BlockSpec((m, k), lambda j: (0, 0)),
            pl.BlockSpec((_BN, k), lambda j: (j, 0)),
            pl.BlockSpec((m, 1), lambda j: (0, 0)),
        ],
        out_specs=pl.BlockSpec((1, 1), lambda j: (0, 0)),
        out_shape=jax.ShapeDtypeStruct((1, 1), jnp.float32),
        scratch_shapes=[
            pltpu.VMEM((m, k), jnp.bfloat16),
            pltpu.VMEM((m, 1), jnp.float32),
            pltpu.VMEM((m, 1), jnp.float32),
        ],
    )(inputs, features, targets.reshape(m, 1))
    return out[0, 0]


# speedup vs baseline: 2.3454x; 1.0473x over previous
import functools

import jax
import jax.numpy as jnp
from jax.experimental import pallas as pl
from jax.experimental.pallas import tpu as pltpu

_TEMP = 0.05
_BN = 1024


def _ce_kernel(x_ref, f_ref, t_ref, out_ref, xb_ref, s_ref, tacc_ref):
    j = pl.program_id(0)
    nj = pl.num_programs(0)
    bn = f_ref.shape[0]

    @pl.when(j == 0)
    def _init():
        x = x_ref[...]
        norm = jnp.sqrt(jnp.sum(x * x, axis=1, keepdims=True))
        xn = x * ((1.0 / _TEMP) / jnp.maximum(norm, 1e-12))
        xb_ref[...] = xn.astype(jnp.bfloat16)
        s_ref[...] = jnp.zeros_like(s_ref)
        tacc_ref[...] = jnp.zeros_like(tacc_ref)

    fb = f_ref[...].astype(jnp.bfloat16)
    logits = jax.lax.dot_general(
        xb_ref[...], fb, (((1,), (1,)), ((), ())),
        preferred_element_type=jnp.float32,
    )
    s_ref[...] += jnp.sum(jnp.exp(logits), axis=1, keepdims=True)
    cols = j * bn + jax.lax.broadcasted_iota(jnp.int32, logits.shape, 1)
    hit = cols == t_ref[...]
    tacc_ref[...] += jnp.sum(jnp.where(hit, logits, 0.0), axis=1, keepdims=True)

    @pl.when(j == nj - 1)
    def _fin():
        per_row = jnp.log(s_ref[...]) - tacc_ref[...]
        out_ref[...] = jnp.sum(per_row, keepdims=True) * (1.0 / per_row.shape[0])


def kernel(epoch, inputs, ema_inputs, part_out, score, targets, features,
           part_features):
    m, k = inputs.shape
    n = features.shape[0]
    out = pl.pallas_call(
        _ce_kernel,
        grid=(n // _BN,),
        in_specs=[
            pl.BlockSpec((m, k), lambda j: (0, 0)),
            pl.BlockSpec((_BN, k), lambda j: (j, 0)),
            pl.BlockSpec((m, 1), lambda j: (0, 0)),
        ],
        out_specs=pl.BlockSpec((1, 1), lambda j: (0, 0)),
        out_shape=jax.ShapeDtypeStruct((1, 1), jnp.float32),
        scratch_shapes=[
            pltpu.VMEM((m, k), jnp.bfloat16),
            pltpu.VMEM((m, 1), jnp.float32),
            pltpu.VMEM((m, 1), jnp.float32),
        ],
    )(inputs, features, targets.reshape(m, 1))
    return out[0, 0]
